# baseline (speedup 1.0000x reference)
import jax
import jax.numpy as jnp
from jax import lax
from jax.experimental import pallas as pl

_N = 2048
_DIM = 256
_KS = [0.9, 0.8, 0.7, 0.6, 0.5, 0.4]
_KVALS = [max(2, int(kf * _N)) for kf in _KS]
_NLEV = 6
_LEVPAD = 8


def _tc_body(h_ref, w0, w1, w2, w3, w4, w5, b_ref, out_ref):
    h = h_ref[...]
    W8 = jnp.concatenate(
        [w0[...], w1[...], w2[...], w3[...], w4[...], w5[...],
         jnp.zeros((_DIM, 2), jnp.float32)], axis=1)
    wt8 = jnp.dot(h, W8, preferred_element_type=jnp.float32) + b_ref[...]
    r = lax.broadcasted_iota(jnp.int32, (_LEVPAD, _LEVPAD), 0)
    c = lax.broadcasted_iota(jnp.int32, (_LEVPAD, _LEVPAD), 1)
    eye8 = (r == c).astype(jnp.float32)
    wtT = lax.dot_general(eye8, wt8, (((1,), (1,)), ((), ())),
                          preferred_element_type=jnp.float32)
    ibits = lax.bitcast_convert_type(wtT, jnp.int32)
    key = jnp.where(ibits >= 0, ibits, ibits ^ jnp.int32(0x7FFFFFFF))
    lev0 = lax.broadcasted_iota(jnp.int32, (_LEVPAD, 1), 0)
    kvec = jnp.where(lev0 < _NLEV, ((9 - lev0) * _N) // 10, _N + 1)

    def step(i, prefix):
        bit = 31 - i
        cand = prefix + (jnp.int32(1) << bit)
        cnt = jnp.sum((key >= cand).astype(jnp.int32), axis=1, keepdims=True)
        return jnp.where(cnt >= kvec, cand, prefix)

    prefix0 = jnp.full((_LEVPAD, 1), jnp.int32(-(2**31)), jnp.int32)
    thr = lax.fori_loop(0, 32, step, prefix0)
    lev = lax.broadcasted_iota(jnp.int32, (_LEVPAD, 1), 0)
    sel = (key >= thr) & (lev < _NLEV)
    scores = 1.0 / (1.0 + jnp.exp(-wtT))
    contrib = jnp.where(sel, scores, 0.0)
    scale = lax.dot_general(contrib, jnp.ones((_LEVPAD, 1), jnp.float32),
                            (((0,), (0,)), ((), ())),
                            preferred_element_type=jnp.float32)
    out_ref[...] = h * scale


def _run_tc(h, Ws, b8, interpret=False):
    return pl.pallas_call(
        _tc_body,
        out_shape=jax.ShapeDtypeStruct((_N, _DIM), jnp.float32),
        interpret=interpret,
    )(h, *Ws, b8)


def kernel(g, h, W0, b0, W1, b1, W2, b2, W3, b3, W4, b4, W5, b5):
    del g
    b8 = jnp.concatenate(
        [b0, b1, b2, b3, b4, b5, jnp.zeros((2,), jnp.float32)]).reshape(1, _LEVPAD)
    return _run_tc(h, (W0, W1, W2, W3, W4, W5), b8)


# speedup vs baseline: 1.4735x; 1.4735x over previous
import jax
import jax.numpy as jnp
from jax import lax
from jax.experimental import pallas as pl

_N = 2048
_DIM = 256
_KS = [0.9, 0.8, 0.7, 0.6, 0.5, 0.4]
_KVALS = [max(2, int(kf * _N)) for kf in _KS]
_NLEV = 6
_LEVPAD = 8


def _tc_body(h_ref, w_ref, b_ref, out_ref):
    h = h_ref[...]
    wt8 = jnp.dot(h, w_ref[...], preferred_element_type=jnp.float32) + b_ref[...]
    r = lax.broadcasted_iota(jnp.int32, (_LEVPAD, _LEVPAD), 0)
    c = lax.broadcasted_iota(jnp.int32, (_LEVPAD, _LEVPAD), 1)
    eye8 = (r == c).astype(jnp.float32)
    wtT = lax.dot_general(eye8, wt8, (((1,), (1,)), ((), ())),
                          precision=lax.Precision.HIGHEST,
                          preferred_element_type=jnp.float32)
    ibits = lax.bitcast_convert_type(wtT, jnp.int32)
    key = jnp.where(ibits >= 0, ibits, ibits ^ jnp.int32(0x7FFFFFFF))
    lev0 = lax.broadcasted_iota(jnp.int32, (_LEVPAD, 1), 0)
    kvec = jnp.where(lev0 < _NLEV, ((9 - lev0) * _N) // 10, _N + 1)

    def step(i, prefix):
        bit = 31 - i
        cand = prefix + (jnp.int32(1) << bit)
        cnt = jnp.sum((key >= cand).astype(jnp.int32), axis=1, keepdims=True)
        return jnp.where(cnt >= kvec, cand, prefix)

    prefix0 = jnp.full((_LEVPAD, 1), jnp.int32(-(2**31)), jnp.int32)
    thr = lax.fori_loop(0, 32, step, prefix0, unroll=True)
    sel = (key >= thr) & (lev0 < _NLEV)
    scores = 1.0 / (1.0 + jnp.exp(-wtT))
    contrib = jnp.where(sel, scores, 0.0)
    scale = lax.dot_general(contrib, jnp.ones((_LEVPAD, 1), jnp.float32),
                            (((0,), (0,)), ((), ())),
                            precision=lax.Precision.HIGHEST,
                            preferred_element_type=jnp.float32)
    out_ref[...] = h * scale


def _run_tc(h, W8, b8, interpret=False):
    return pl.pallas_call(
        _tc_body,
        out_shape=jax.ShapeDtypeStruct((_N, _DIM), jnp.float32),
        interpret=interpret,
    )(h, W8, b8)


def kernel(g, h, W0, b0, W1, b1, W2, b2, W3, b3, W4, b4, W5, b5):
    del g
    W8 = jnp.concatenate(
        [W0, W1, W2, W3, W4, W5, jnp.zeros((_DIM, 2), jnp.float32)], axis=1)
    b8 = jnp.concatenate(
        [b0, b1, b2, b3, b4, b5, jnp.zeros((2,), jnp.float32)]).reshape(1, _LEVPAD)
    return _run_tc(h, W8, b8)
